# Initial kernel scaffold; baseline (speedup 1.0000x reference)
import jax
import jax.numpy as jnp
from jax.experimental import pallas as pl


def kernel(x, edge_index, edge_attr, edge_weight, Wl1, bl1, Wr1, L1W, L1b, Wl2, bl2, Wr2, L2W, L2b):
    raise NotImplementedError("write your pallas kernel here")


# speedup vs baseline: 7.2440x; 7.2440x over previous
import functools

import jax
import jax.numpy as jnp
from jax import lax
from jax.experimental import pallas as pl
from jax.experimental.pallas import tpu as pltpu
from jax.experimental.pallas import tpu_sc as plsc

N = 10000
E = 320000
D = 128
NP = 10240
NS = 16
EPW = E // NS
K = 80
G = 25
NG = EPW // (K * G)
RPT = NP // NS


def _sc_scatter(xg, src4, dst4, with_deg):
    mesh = plsc.VectorSubcoreMesh(
        core_axis_name="c", subcore_axis_name="s", num_cores=1)
    out_type = [jax.ShapeDtypeStruct((NP, D), jnp.float32)]
    if with_deg:
        out_type.append(jax.ShapeDtypeStruct((NS, NP), jnp.float32))

    @functools.partial(
        pl.kernel,
        out_type=out_type,
        mesh=mesh,
        compiler_params=pltpu.CompilerParams(needs_layout_passes=False),
        scratch_types=[
            pltpu.VMEM((G, K), jnp.int32),
            pltpu.VMEM((G, K), jnp.int32),
            pltpu.VMEM((G, K), jnp.int32),
            pltpu.VMEM((G, K), jnp.int32),
            pltpu.VMEM((K, D), jnp.float32),
            pltpu.VMEM((K, D), jnp.float32),
            pltpu.VMEM((NP,), jnp.float32),
            pltpu.VMEM_SHARED((NP, D), jnp.float32),
            pltpu.SemaphoreType.DMA,
            pltpu.SemaphoreType.DMA,
            pltpu.SemaphoreType.DMA,
        ],
    )
    def scat(xg_hbm, src_hbm, dst_hbm, agg_hbm, *rest):
        if with_deg:
            deg_hbm = rest[0]
            rest = rest[1:]
        (srcA, dstA, srcB, dstB, rows0, rows1, degl, acc,
         sem0, sem1, semi) = rest
        rows = (rows0, rows1)
        sems = (sem0, sem1)
        slots = ((srcA, dstA), (srcB, dstB))
        sid = lax.axis_index("s")

        zero16 = jnp.zeros((16,), jnp.float32)

        def zrow(i, carry):
            for c0 in range(0, D, 16):
                rows0[i, pl.ds(c0, 16)] = zero16
            return carry

        lax.fori_loop(0, K, zrow, 0)
        for k in range(RPT // K):
            pltpu.sync_copy(rows0, acc.at[pl.ds(sid * RPT + k * K, K)])
        if with_deg:
            def zdeg(i, carry):
                degl[pl.ds(i * 16, 16)] = zero16
                return carry

            lax.fori_loop(0, NP // 16, zdeg, 0)
        plsc.subcore_barrier()

        def idx_start(g, slot):
            src_s, dst_s = slots[slot]
            pltpu.make_async_copy(src_hbm.at[sid, g], src_s, semi).start()
            pltpu.make_async_copy(dst_hbm.at[sid, g], dst_s, semi).start()

        def idx_wait():
            pltpu.make_async_copy(src_hbm.at[sid, 0], srcA, semi).wait()
            pltpu.make_async_copy(dst_hbm.at[sid, 0], dstA, semi).wait()

        def g_start(slot, ct, buf, sem):
            pltpu.make_async_copy(
                xg_hbm.at[slots[slot][0].at[ct]], buf, sem).start()

        def g_wait(buf, sem):
            pltpu.make_async_copy(xg_hbm.at[srcA.at[0]], buf, sem).wait()

        idx_start(0, 0)
        idx_wait()
        g_start(0, 0, rows0, sem0)

        def pair_body(j, carry):
            for half in range(2):
                g = 2 * j + half
                slot = half
                nslot = 1 - half
                have_next = g + 1 <= NG - 1
                par0 = half * (G % 2)
                for ct in range(G):
                    b = (par0 + ct) % 2
                    bn = (par0 + ct + 1) % 2
                    if ct == 0:
                        @pl.when(have_next)
                        def _():
                            idx_start(g + 1, nslot)
                    if ct == G - 2:
                        @pl.when(have_next)
                        def _():
                            idx_wait()
                    if ct < G - 1:
                        g_start(slot, ct + 1, rows[bn], sems[bn])
                    else:
                        @pl.when(have_next)
                        def _():
                            g_start(nslot, 0, rows[bn], sems[bn])
                    if with_deg:
                        dst_s = slots[slot][1]
                        for k in range(K // 16):
                            idx = dst_s[ct, pl.ds(k * 16, 16)]
                            cnt, last = plsc.scan_count(idx)
                            plsc.addupdate_scatter(
                                degl, [idx], cnt.astype(jnp.float32),
                                mask=last)
                    g_wait(rows[b], sems[b])
                    pltpu.sync_copy(
                        rows[b], acc.at[slots[slot][1].at[ct]], add=True)
            return carry

        lax.fori_loop(0, NG // 2, pair_body, 0)
        if with_deg:
            pltpu.sync_copy(degl, deg_hbm.at[sid])
        plsc.subcore_barrier()

        pltpu.sync_copy(acc.at[pl.ds(sid * RPT, RPT)],
                        agg_hbm.at[pl.ds(sid * RPT, RPT)])

    res = scat(xg, src4, dst4)
    if with_deg:
        return res[0], res[1]
    return res[0]


def _dense_layer(agg, deg, xg, WlT, WcT, b, relu):
    R = 256

    def body(agg_ref, deg_ref, x_ref, wl_ref, wc_ref, b_ref, o_ref):
        a = agg_ref[...]
        ones = jnp.ones((NS, 1), jnp.float32)
        deg_col = lax.dot_general(
            deg_ref[...], ones, (((0,), (0,)), ((), ())),
            preferred_element_type=jnp.float32)
        mean = a / jnp.maximum(deg_col, 1.0)
        h = jnp.dot(mean, wl_ref[...], preferred_element_type=jnp.float32)
        h = h + jnp.dot(x_ref[...], wc_ref[...],
                        preferred_element_type=jnp.float32)
        h = h + b_ref[...]
        if relu:
            h = jnp.maximum(h, 0.0)
        o_ref[...] = h

    return pl.pallas_call(
        body,
        grid=(NP // R,),
        in_specs=[
            pl.BlockSpec((R, D), lambda i: (i, 0)),
            pl.BlockSpec((NS, R), lambda i: (0, i)),
            pl.BlockSpec((R, D), lambda i: (i, 0)),
            pl.BlockSpec((D, D), lambda i: (0, 0)),
            pl.BlockSpec((D, D), lambda i: (0, 0)),
            pl.BlockSpec((1, D), lambda i: (0, 0)),
        ],
        out_specs=pl.BlockSpec((R, D), lambda i: (i, 0)),
        out_shape=jax.ShapeDtypeStruct((NP, D), jnp.float32),
    )(agg, deg, xg, WlT, WcT, b)


def kernel(x, edge_index, edge_attr, edge_weight,
           Wl1, bl1, Wr1, L1W, L1b, Wl2, bl2, Wr2, L2W, L2b):
    src4 = edge_index[0].reshape(NS, NG, G, K)
    dst4 = edge_index[1].reshape(NS, NG, G, K)

    xg = jnp.pad(x, ((0, NP - N), (0, 0)))

    Wl1T = Wl1.T
    Wc1T = (Wr1 + L1W).T
    b1 = (bl1 + L1b).reshape(1, D)
    Wl2T = Wl2.T
    Wc2T = (Wr2 + L2W).T
    b2 = (bl2 + L2b).reshape(1, D)

    agg1, deg = _sc_scatter(xg, src4, dst4, with_deg=True)
    hg = _dense_layer(agg1, deg, xg, Wl1T, Wc1T, b1, relu=True)
    agg2 = _sc_scatter(hg, src4, dst4, with_deg=False)
    out = _dense_layer(agg2, deg, hg, Wl2T, Wc2T, b2, relu=False)
    return out[:N]
